# Initial kernel scaffold; baseline (speedup 1.0000x reference)
import jax
import jax.numpy as jnp
from jax.experimental import pallas as pl


def kernel(xyz, features, W1, b1, W2, b2, Wl, bl):
    raise NotImplementedError("write your pallas kernel here")


# speedup vs baseline: 9.1481x; 9.1481x over previous
import functools

import jax
import jax.numpy as jnp
from jax import lax
from jax.experimental import pallas as pl
from jax.experimental.pallas import tpu as pltpu
from jax.experimental.pallas import tpu_sc as plsc

B, N, C_IN, C_OUT, K = 4, 4096, 64, 128, 16
BS = B * N
D_PAD = 80
TD = 128
TS = 256
NW = 32
PW = BS // NW
CH = 8


def _leaky(x):
    return jnp.where(x >= 0, x, 0.1 * x)


def _topk_body(xt_ref, xn_ref, out_ref):
    b = pl.program_id(0)
    s2 = None
    n2 = None
    cross = None
    for c in range(3):
        a = xt_ref[0, :, c:c + 1]
        v = xn_ref[0, c:c + 1, :]
        sa = a * a
        sv = v * v
        s2 = sa if s2 is None else s2 + sa
        n2 = sv if n2 is None else n2 + sv
        ab = a.astype(jnp.bfloat16).astype(jnp.float32)
        vb = v.astype(jnp.bfloat16).astype(jnp.float32)
        p = ab * vb
        cross = p if cross is None else cross + p
    d2 = (s2 + n2) - 2.0 * cross
    bits = lax.bitcast_convert_type(d2, jnp.int32)
    pk = jnp.bitwise_xor(
        bits, jnp.bitwise_and(jnp.right_shift(bits, 31), jnp.int32(0x7FFFFFFF)))
    col = lax.broadcasted_iota(jnp.int32, (TS, N), 1)
    lane16 = lax.broadcasted_iota(jnp.int32, (TS, K), 1)
    acc0 = jnp.zeros((TS, K), jnp.int32)
    big = jnp.int32(0x7FFFFFFF)

    def it(i, carry):
        pk, acc = carry
        m = jnp.min(pk, axis=1, keepdims=True)
        loc = jnp.min(jnp.where(pk == m, col, big), axis=1, keepdims=True)
        acc = jnp.where(lane16 == i, loc, acc)
        pk = jnp.where(col == loc, big, pk)
        return pk, acc

    _, acc = lax.fori_loop(0, K, it, (pk, acc0))
    out_ref[0] = acc + b * N


def _topk_call(xyz_t, xyz_pad):
    return pl.pallas_call(
        _topk_body,
        grid=(B, N // TS),
        in_specs=[
            pl.BlockSpec((1, TS, 3), lambda b, t: (b, t, 0)),
            pl.BlockSpec((1, 8, N), lambda b, t: (b, 0, 0)),
        ],
        out_specs=pl.BlockSpec((1, TS, K), lambda b, t: (b, t, 0)),
        out_shape=jax.ShapeDtypeStruct((B, N, K), jnp.int32),
    )(xyz_t, xyz_pad)


def _sc_agg_body(idx_hbm, table_hbm, w1_hbm, b1_hbm, w2_hbm, b2_hbm, out_hbm,
                 idx_v, rows_v, cent_v, t_v, w_v, xp_v,
                 w1_v, b1_v, w2_v, b2_v, sem):
    cid = lax.axis_index("c")
    sid = lax.axis_index("s")
    wid = sid * 2 + cid
    pltpu.sync_copy(w1_hbm, w1_v)
    pltpu.sync_copy(b1_hbm, b1_v)
    pltpu.sync_copy(w2_hbm, w2_v)
    pltpu.sync_copy(b2_hbm, b2_v)
    li = lax.iota(jnp.int32, 16)
    w1s = [[w1_v[i, :][c] for c in range(3)] for i in range(8)]
    b1v = b1_v[:]
    b1s = [b1v[i] for i in range(8)]
    w2s = [[w2_v[m, :][i] for i in range(8)] for m in range(K)]
    b2v = b2_v[:]
    b2s = [b2v[m] for m in range(K)]

    def chunk_body(ch, carry):
        pbase = wid * PW + ch * CH
        pltpu.sync_copy(idx_hbm.at[pl.ds(pbase * K, CH * K)], idx_v)
        pltpu.async_copy(table_hbm.at[idx_v], rows_v, sem).wait()
        pltpu.sync_copy(table_hbm.at[pl.ds(pbase, CH)], cent_v)

        def pt_body(p, carry2):
            rbase = p * K
            cvec = cent_v[p, 0:16]
            cx = cvec[0]
            cy = cvec[1]
            cz = cvec[2]
            for j in range(K):
                fh = rows_v[rbase + j, 0:16]
                plsc.store_scatter(xp_v, [li * 16 + j], fh)
            relx = xp_v[0:16] - cx
            rely = xp_v[16:32] - cy
            relz = xp_v[32:48] - cz
            hs = []
            for i in range(8):
                h = relx * w1s[i][0] + rely * w1s[i][1] + relz * w1s[i][2] + b1s[i]
                hs.append(_leaky(h))
            for m in range(K):
                hacc = hs[0] * w2s[m][0]
                for i in range(1, 8):
                    hacc = hacc + hs[i] * w2s[m][i]
                w_v[m, :] = _leaky(hacc + b2s[m])
            for half in range(2):
                accs = [[jnp.zeros((16,), jnp.float32) for _ in range(5)]
                        for _ in range(8)]
                for j in range(K):
                    fj = [rows_v[rbase + j, cc * 16:(cc + 1) * 16]
                          for cc in range(5)]
                    wcol = [w_v[half * 8 + mm, :][j] for mm in range(8)]
                    for mm in range(8):
                        ws = wcol[mm]
                        for cc in range(5):
                            accs[mm][cc] = accs[mm][cc] + fj[cc] * ws
                for mm in range(8):
                    for cc in range(5):
                        t_v[p, half * 8 + mm, cc * 16:(cc + 1) * 16] = accs[mm][cc]
            return carry2

        lax.fori_loop(0, CH, pt_body, 0)
        pltpu.sync_copy(t_v, out_hbm.at[pl.ds(pbase, CH)])
        return carry

    lax.fori_loop(0, PW // CH, chunk_body, 0)


def _sc_agg(idx_flat, table, W1, b1, W2, b2):
    mesh = plsc.VectorSubcoreMesh(core_axis_name="c", subcore_axis_name="s",
                                  num_cores=2, num_subcores=16)
    kern = pl.kernel(
        _sc_agg_body,
        out_type=jax.ShapeDtypeStruct((BS, K, D_PAD), jnp.float32),
        mesh=mesh,
        compiler_params=pltpu.CompilerParams(needs_layout_passes=False),
        scratch_types=[
            pltpu.VMEM((CH * K,), jnp.int32),
            pltpu.VMEM((CH * K, TD), jnp.float32),
            pltpu.VMEM((CH, TD), jnp.float32),
            pltpu.VMEM((CH, K, D_PAD), jnp.float32),
            pltpu.VMEM((K, K), jnp.float32),
            pltpu.VMEM((256,), jnp.float32),
            pltpu.VMEM((8, 16), jnp.float32),
            pltpu.VMEM((16,), jnp.float32),
            pltpu.VMEM((K, 16), jnp.float32),
            pltpu.VMEM((K,), jnp.float32),
            pltpu.SemaphoreType.DMA,
        ],
    )
    return kern(idx_flat, table, W1, b1, W2, b2)


RT = 1024


def _final_body(t_ref, w_ref, b_ref, out_ref):
    acc = jnp.dot(t_ref[...], w_ref[...], preferred_element_type=jnp.float32)
    acc = acc + b_ref[0:1, :]
    out_ref[...] = _leaky(acc)


def _final_call(t_flat, wp, bl2):
    return pl.pallas_call(
        _final_body,
        grid=(BS // RT,),
        in_specs=[
            pl.BlockSpec((RT, K * D_PAD), lambda i: (i, 0)),
            pl.BlockSpec((K * D_PAD, C_OUT), lambda i: (0, 0)),
            pl.BlockSpec((1, C_OUT), lambda i: (0, 0)),
        ],
        out_specs=pl.BlockSpec((RT, C_OUT), lambda i: (i, 0)),
        out_shape=jax.ShapeDtypeStruct((BS, C_OUT), jnp.float32),
    )(t_flat, wp, bl2)


@jax.jit
def kernel(xyz, features, W1, b1, W2, b2, Wl, bl):
    xyz_t = jnp.transpose(xyz, (0, 2, 1))
    xyz_pad = jnp.pad(xyz, ((0, 0), (0, 5), (0, 0)))
    idx = _topk_call(xyz_t, xyz_pad)
    idx_flat = idx.reshape(BS * K)

    feats_t = jnp.transpose(features, (0, 2, 1))
    table = jnp.concatenate([xyz_t, feats_t], axis=2)
    table = jnp.pad(table, ((0, 0), (0, 0), (0, TD - 3 - C_IN)))
    table = table.reshape(BS, TD)

    w1p = jnp.pad(W1, ((0, 0), (0, 13)))
    b1p = jnp.pad(b1, (0, 8))
    w2p = jnp.pad(W2, ((0, 0), (0, 8)))
    t_agg = _sc_agg(idx_flat, table, w1p, b1p, w2p, b2)

    wl3 = Wl.reshape(C_OUT, K, 3 + C_IN)
    wl3 = jnp.pad(wl3, ((0, 0), (0, 0), (0, D_PAD - 3 - C_IN)))
    wp = wl3.reshape(C_OUT, K * D_PAD).T
    out = _final_call(t_agg.reshape(BS, K * D_PAD), wp, bl.reshape(1, C_OUT))
    out = out.reshape(B, N, C_OUT)
    return jnp.transpose(out, (0, 2, 1))


# speedup vs baseline: 9.9431x; 1.0869x over previous
import functools

import jax
import jax.numpy as jnp
from jax import lax
from jax.experimental import pallas as pl
from jax.experimental.pallas import tpu as pltpu
from jax.experimental.pallas import tpu_sc as plsc

B, N, C_IN, C_OUT, K = 4, 4096, 64, 128, 16
BS = B * N
D_PAD = 80
TD = 128
TS = 256
NW = 32
PW = BS // NW
CH = 8


def _leaky(x):
    return jnp.where(x >= 0, x, 0.1 * x)


def _topk_body(xt_ref, xn_ref, out_ref):
    b = pl.program_id(0)
    s2 = None
    n2 = None
    cross = None
    for c in range(3):
        a = xt_ref[0, :, c:c + 1]
        v = xn_ref[0, c:c + 1, :]
        sa = a * a
        sv = v * v
        s2 = sa if s2 is None else s2 + sa
        n2 = sv if n2 is None else n2 + sv
        ab = a.astype(jnp.bfloat16).astype(jnp.float32)
        vb = v.astype(jnp.bfloat16).astype(jnp.float32)
        p = ab * vb
        cross = p if cross is None else cross + p
    d2 = (s2 + n2) - 2.0 * cross
    bits = lax.bitcast_convert_type(d2, jnp.int32)
    pk = jnp.bitwise_xor(
        bits, jnp.bitwise_and(jnp.right_shift(bits, 31), jnp.int32(0x7FFFFFFF)))
    col = lax.broadcasted_iota(jnp.int32, (TS, N), 1)
    lane16 = lax.broadcasted_iota(jnp.int32, (TS, K), 1)
    acc0 = jnp.zeros((TS, K), jnp.int32)
    big = jnp.int32(0x7FFFFFFF)

    def it(i, carry):
        pk, acc = carry
        m = jnp.min(pk, axis=1, keepdims=True)
        hit = pk == m
        loc = jnp.min(jnp.where(hit, col, big), axis=1, keepdims=True)
        acc = jnp.where(lane16 == i, loc, acc)
        pk = jnp.where(hit, big, pk)
        return pk, acc

    _, acc = lax.fori_loop(0, K, it, (pk, acc0))
    out_ref[0] = acc + b * N


def _topk_call(xyz_t, xyz_pad):
    return pl.pallas_call(
        _topk_body,
        grid=(B, N // TS),
        in_specs=[
            pl.BlockSpec((1, TS, 3), lambda b, t: (b, t, 0)),
            pl.BlockSpec((1, 8, N), lambda b, t: (b, 0, 0)),
        ],
        out_specs=pl.BlockSpec((1, TS, K), lambda b, t: (b, t, 0)),
        out_shape=jax.ShapeDtypeStruct((B, N, K), jnp.int32),
    )(xyz_t, xyz_pad)


def _sc_agg_body(idx_hbm, table_hbm, w1_hbm, b1_hbm, w2_hbm, b2_hbm, out_hbm,
                 idx_v, rows_v, cent_v, t_v, w_v, xp_v,
                 w1_v, b1_v, w2_v, b2_v, sem):
    cid = lax.axis_index("c")
    sid = lax.axis_index("s")
    wid = sid * 2 + cid
    pltpu.sync_copy(w1_hbm, w1_v)
    pltpu.sync_copy(b1_hbm, b1_v)
    pltpu.sync_copy(w2_hbm, w2_v)
    pltpu.sync_copy(b2_hbm, b2_v)
    li = lax.iota(jnp.int32, 16)
    w1s = [[w1_v[i, :][c] for c in range(3)] for i in range(8)]
    b1v = b1_v[:]
    b1s = [b1v[i] for i in range(8)]
    w2s = [[w2_v[m, :][i] for i in range(8)] for m in range(K)]
    b2v = b2_v[:]
    b2s = [b2v[m] for m in range(K)]

    def chunk_body(ch, carry):
        pbase = wid * PW + ch * CH
        pltpu.sync_copy(idx_hbm.at[pl.ds(pbase * K, CH * K)], idx_v)
        pltpu.async_copy(table_hbm.at[idx_v], rows_v, sem).wait()
        pltpu.sync_copy(table_hbm.at[pl.ds(pbase, CH)], cent_v)

        def pt_body(p, carry2):
            rbase = p * K
            cvec = cent_v[p, 0:16]
            cx = cvec[0]
            cy = cvec[1]
            cz = cvec[2]
            for j in range(K):
                fh = rows_v[rbase + j, 0:16]
                plsc.store_scatter(xp_v, [li * 16 + j], fh)
            relx = xp_v[0:16] - cx
            rely = xp_v[16:32] - cy
            relz = xp_v[32:48] - cz
            hs = []
            for i in range(8):
                h = relx * w1s[i][0] + rely * w1s[i][1] + relz * w1s[i][2] + b1s[i]
                hs.append(_leaky(h))
            for m in range(K):
                hacc = hs[0] * w2s[m][0]
                for i in range(1, 8):
                    hacc = hacc + hs[i] * w2s[m][i]
                w_v[m, :] = _leaky(hacc + b2s[m])
            for half in range(2):
                accs = [[jnp.zeros((16,), jnp.float32) for _ in range(5)]
                        for _ in range(8)]
                wrows = [w_v[half * 8 + mm, :] for mm in range(8)]
                for j in range(K):
                    fj = [rows_v[rbase + j, cc * 16:(cc + 1) * 16]
                          for cc in range(5)]
                    for mm in range(8):
                        ws = wrows[mm][j]
                        for cc in range(5):
                            accs[mm][cc] = accs[mm][cc] + fj[cc] * ws
                for mm in range(8):
                    for cc in range(5):
                        t_v[p, half * 8 + mm, cc * 16:(cc + 1) * 16] = accs[mm][cc]
            return carry2

        lax.fori_loop(0, CH, pt_body, 0)
        pltpu.sync_copy(t_v, out_hbm.at[pl.ds(pbase, CH)])
        return carry

    lax.fori_loop(0, PW // CH, chunk_body, 0)


def _sc_agg(idx_flat, table, W1, b1, W2, b2):
    mesh = plsc.VectorSubcoreMesh(core_axis_name="c", subcore_axis_name="s",
                                  num_cores=2, num_subcores=16)
    kern = pl.kernel(
        _sc_agg_body,
        out_type=jax.ShapeDtypeStruct((BS, K, D_PAD), jnp.float32),
        mesh=mesh,
        compiler_params=pltpu.CompilerParams(needs_layout_passes=False),
        scratch_types=[
            pltpu.VMEM((CH * K,), jnp.int32),
            pltpu.VMEM((CH * K, TD), jnp.float32),
            pltpu.VMEM((CH, TD), jnp.float32),
            pltpu.VMEM((CH, K, D_PAD), jnp.float32),
            pltpu.VMEM((K, K), jnp.float32),
            pltpu.VMEM((256,), jnp.float32),
            pltpu.VMEM((8, 16), jnp.float32),
            pltpu.VMEM((16,), jnp.float32),
            pltpu.VMEM((K, 16), jnp.float32),
            pltpu.VMEM((K,), jnp.float32),
            pltpu.SemaphoreType.DMA,
        ],
    )
    return kern(idx_flat, table, W1, b1, W2, b2)


RT = 1024


def _final_body(t_ref, w_ref, b_ref, out_ref):
    acc = jnp.dot(t_ref[...], w_ref[...], preferred_element_type=jnp.float32)
    acc = acc + b_ref[0:1, :]
    out_ref[...] = _leaky(acc)


def _final_call(t_flat, wp, bl2):
    return pl.pallas_call(
        _final_body,
        grid=(BS // RT,),
        in_specs=[
            pl.BlockSpec((RT, K * D_PAD), lambda i: (i, 0)),
            pl.BlockSpec((K * D_PAD, C_OUT), lambda i: (0, 0)),
            pl.BlockSpec((1, C_OUT), lambda i: (0, 0)),
        ],
        out_specs=pl.BlockSpec((RT, C_OUT), lambda i: (i, 0)),
        out_shape=jax.ShapeDtypeStruct((BS, C_OUT), jnp.float32),
    )(t_flat, wp, bl2)


@jax.jit
def kernel(xyz, features, W1, b1, W2, b2, Wl, bl):
    xyz_t = jnp.transpose(xyz, (0, 2, 1))
    xyz_pad = jnp.pad(xyz, ((0, 0), (0, 5), (0, 0)))
    idx = _topk_call(xyz_t, xyz_pad)
    idx_flat = idx.reshape(BS * K)

    feats_t = jnp.transpose(features, (0, 2, 1))
    table = jnp.concatenate([xyz_t, feats_t], axis=2)
    table = jnp.pad(table, ((0, 0), (0, 0), (0, TD - 3 - C_IN)))
    table = table.reshape(BS, TD)

    w1p = jnp.pad(W1, ((0, 0), (0, 13)))
    b1p = jnp.pad(b1, (0, 8))
    w2p = jnp.pad(W2, ((0, 0), (0, 8)))
    t_agg = _sc_agg(idx_flat, table, w1p, b1p, w2p, b2)

    wl3 = Wl.reshape(C_OUT, K, 3 + C_IN)
    wl3 = jnp.pad(wl3, ((0, 0), (0, 0), (0, D_PAD - 3 - C_IN)))
    wp = wl3.reshape(C_OUT, K * D_PAD).T
    out = _final_call(t_agg.reshape(BS, K * D_PAD), wp, bl.reshape(1, C_OUT))
    out = out.reshape(B, N, C_OUT)
    return jnp.transpose(out, (0, 2, 1))


# speedup vs baseline: 11.5808x; 1.1647x over previous
import functools

import jax
import jax.numpy as jnp
from jax import lax
from jax.experimental import pallas as pl
from jax.experimental.pallas import tpu as pltpu
from jax.experimental.pallas import tpu_sc as plsc

B, N, C_IN, C_OUT, K = 4, 4096, 64, 128, 16
BS = B * N
D_PAD = 80
TD = 128
TS = 256
NW = 32
PWB = N // NW
CH = 8


def _leaky(x):
    return jnp.where(x >= 0, x, 0.1 * x)


def _topk_body(xt_ref, xn_ref, out_ref, *, b):
    s2 = None
    n2 = None
    cross = None
    for c in range(3):
        a = xt_ref[0, :, c:c + 1]
        v = xn_ref[0, c:c + 1, :]
        sa = a * a
        sv = v * v
        s2 = sa if s2 is None else s2 + sa
        n2 = sv if n2 is None else n2 + sv
        ab = a.astype(jnp.bfloat16).astype(jnp.float32)
        vb = v.astype(jnp.bfloat16).astype(jnp.float32)
        p = ab * vb
        cross = p if cross is None else cross + p
    d2 = (s2 + n2) - 2.0 * cross
    bits = lax.bitcast_convert_type(d2, jnp.int32)
    pk = jnp.bitwise_xor(
        bits, jnp.bitwise_and(jnp.right_shift(bits, 31), jnp.int32(0x7FFFFFFF)))
    col = lax.broadcasted_iota(jnp.int32, (TS, N), 1)
    lane16 = lax.broadcasted_iota(jnp.int32, (TS, K), 1)
    acc0 = jnp.zeros((TS, K), jnp.int32)
    big = jnp.int32(0x7FFFFFFF)

    def it(i, carry):
        pk, acc = carry
        m = jnp.min(pk, axis=1, keepdims=True)
        hit = pk == m
        loc = jnp.min(jnp.where(hit, col, big), axis=1, keepdims=True)
        acc = jnp.where(lane16 == i, loc, acc)
        pk = jnp.where(hit, big, pk)
        return pk, acc

    _, acc = lax.fori_loop(0, K, it, (pk, acc0))
    out_ref[0] = acc + b * N


def _topk_call(xyz_t, xyz_pad, b):
    return pl.pallas_call(
        functools.partial(_topk_body, b=b),
        grid=(N // TS,),
        in_specs=[
            pl.BlockSpec((1, TS, 3), lambda t, _b=b: (_b, t, 0)),
            pl.BlockSpec((1, 8, N), lambda t, _b=b: (_b, 0, 0)),
        ],
        out_specs=pl.BlockSpec((1, TS, K), lambda t: (0, t, 0)),
        out_shape=jax.ShapeDtypeStruct((1, N, K), jnp.int32),
    )(xyz_t, xyz_pad)


def _sc_agg_body(idx_hbm, table_hbm, w1_hbm, b1_hbm, w2_hbm, b2_hbm, out_hbm,
                 idx_v, rows_v, cent_v, t_v, w_v, xp_v,
                 w1_v, b1_v, w2_v, b2_v, sem, *, base):
    cid = lax.axis_index("c")
    sid = lax.axis_index("s")
    wid = sid * 2 + cid
    pltpu.sync_copy(w1_hbm, w1_v)
    pltpu.sync_copy(b1_hbm, b1_v)
    pltpu.sync_copy(w2_hbm, w2_v)
    pltpu.sync_copy(b2_hbm, b2_v)
    li = lax.iota(jnp.int32, 16)
    w1s = [[w1_v[i, :][c] for c in range(3)] for i in range(8)]
    b1v = b1_v[:]
    b1s = [b1v[i] for i in range(8)]
    w2s = [[w2_v[m, :][i] for i in range(8)] for m in range(K)]
    b2v = b2_v[:]
    b2s = [b2v[m] for m in range(K)]

    def chunk_body(ch, carry):
        pbase = wid * PWB + ch * CH
        pltpu.sync_copy(idx_hbm.at[pl.ds(pbase * K, CH * K)], idx_v)
        pltpu.async_copy(table_hbm.at[idx_v], rows_v, sem).wait()
        pltpu.sync_copy(table_hbm.at[pl.ds(base + pbase, CH)], cent_v)

        def pt_body(p, carry2):
            rbase = p * K
            cvec = cent_v[p, 0:16]
            cx = cvec[0]
            cy = cvec[1]
            cz = cvec[2]
            for j in range(K):
                fh = rows_v[rbase + j, 0:16]
                plsc.store_scatter(xp_v, [li * 16 + j], fh)
            relx = xp_v[0:16] - cx
            rely = xp_v[16:32] - cy
            relz = xp_v[32:48] - cz
            hs = []
            for i in range(8):
                h = relx * w1s[i][0] + rely * w1s[i][1] + relz * w1s[i][2] + b1s[i]
                hs.append(_leaky(h))
            for m in range(K):
                hacc = hs[0] * w2s[m][0]
                for i in range(1, 8):
                    hacc = hacc + hs[i] * w2s[m][i]
                w_v[m, :] = _leaky(hacc + b2s[m])
            for half in range(2):
                accs = [[jnp.zeros((16,), jnp.float32) for _ in range(5)]
                        for _ in range(8)]
                wrows = [w_v[half * 8 + mm, :] for mm in range(8)]
                for j in range(K):
                    fj = [rows_v[rbase + j, cc * 16:(cc + 1) * 16]
                          for cc in range(5)]
                    for mm in range(8):
                        ws = wrows[mm][j]
                        for cc in range(5):
                            accs[mm][cc] = accs[mm][cc] + fj[cc] * ws
                for mm in range(8):
                    for cc in range(5):
                        t_v[p, half * 8 + mm, cc * 16:(cc + 1) * 16] = accs[mm][cc]
            return carry2

        lax.fori_loop(0, CH, pt_body, 0)
        pltpu.sync_copy(t_v, out_hbm.at[pl.ds(pbase, CH)])
        return carry

    lax.fori_loop(0, PWB // CH, chunk_body, 0)


def _sc_agg(idx_flat, table, W1, b1, W2, b2, base):
    mesh = plsc.VectorSubcoreMesh(core_axis_name="c", subcore_axis_name="s",
                                  num_cores=2, num_subcores=16)
    kern = pl.kernel(
        functools.partial(_sc_agg_body, base=base),
        out_type=jax.ShapeDtypeStruct((N, K, D_PAD), jnp.float32),
        mesh=mesh,
        compiler_params=pltpu.CompilerParams(needs_layout_passes=False),
        scratch_types=[
            pltpu.VMEM((CH * K,), jnp.int32),
            pltpu.VMEM((CH * K, TD), jnp.float32),
            pltpu.VMEM((CH, TD), jnp.float32),
            pltpu.VMEM((CH, K, D_PAD), jnp.float32),
            pltpu.VMEM((K, K), jnp.float32),
            pltpu.VMEM((256,), jnp.float32),
            pltpu.VMEM((8, 16), jnp.float32),
            pltpu.VMEM((16,), jnp.float32),
            pltpu.VMEM((K, 16), jnp.float32),
            pltpu.VMEM((K,), jnp.float32),
            pltpu.SemaphoreType.DMA,
        ],
    )
    return kern(idx_flat, table, W1, b1, W2, b2)


RT = 1024


def _final_body(t_ref, w_ref, b_ref, out_ref):
    acc = jnp.dot(t_ref[...], w_ref[...], preferred_element_type=jnp.float32)
    acc = acc + b_ref[0:1, :]
    out_ref[...] = _leaky(acc)


def _final_call(t_flat, wp, bl2):
    return pl.pallas_call(
        _final_body,
        grid=(BS // RT,),
        in_specs=[
            pl.BlockSpec((RT, K * D_PAD), lambda i: (i, 0)),
            pl.BlockSpec((K * D_PAD, C_OUT), lambda i: (0, 0)),
            pl.BlockSpec((1, C_OUT), lambda i: (0, 0)),
        ],
        out_specs=pl.BlockSpec((RT, C_OUT), lambda i: (i, 0)),
        out_shape=jax.ShapeDtypeStruct((BS, C_OUT), jnp.float32),
    )(t_flat, wp, bl2)


@jax.jit
def kernel(xyz, features, W1, b1, W2, b2, Wl, bl):
    xyz_t = jnp.transpose(xyz, (0, 2, 1))
    xyz_pad = jnp.pad(xyz, ((0, 0), (0, 5), (0, 0)))

    feats_t = jnp.transpose(features, (0, 2, 1))
    table = jnp.concatenate([xyz_t, feats_t], axis=2)
    table = jnp.pad(table, ((0, 0), (0, 0), (0, TD - 3 - C_IN)))
    table = table.reshape(BS, TD)

    w1p = jnp.pad(W1, ((0, 0), (0, 13)))
    b1p = jnp.pad(b1, (0, 8))
    w2p = jnp.pad(W2, ((0, 0), (0, 8)))

    t_parts = []
    for b in range(B):
        idx_b = _topk_call(xyz_t, xyz_pad, b)
        t_parts.append(
            _sc_agg(idx_b.reshape(N * K), table, w1p, b1p, w2p, b2, b * N))
    t_agg = jnp.concatenate(t_parts, axis=0)

    wl3 = Wl.reshape(C_OUT, K, 3 + C_IN)
    wl3 = jnp.pad(wl3, ((0, 0), (0, 0), (0, D_PAD - 3 - C_IN)))
    wp = wl3.reshape(C_OUT, K * D_PAD).T
    out = _final_call(t_agg.reshape(BS, K * D_PAD), wp, bl.reshape(1, C_OUT))
    out = out.reshape(B, N, C_OUT)
    return jnp.transpose(out, (0, 2, 1))
